# Initial kernel scaffold; baseline (speedup 1.0000x reference)
import jax
import jax.numpy as jnp
from jax.experimental import pallas as pl


def kernel(input_ids, embeds, context_lengths, spec_W, spec_emb, spec_W_out, model_emb, model_W, lm_head_W):
    raise NotImplementedError("write your pallas kernel here")


# speedup vs baseline: 40.4263x; 40.4263x over previous
import functools

import jax
import jax.numpy as jnp
from jax import lax
from jax.experimental import pallas as pl
from jax.experimental.pallas import tpu as pltpu

B = 32
D = 1024
V = 32000
NP = 3
TOPK = 5
THRESHES = (5, 3, 2)
NADDS = NP + 1
VT = 3200
NT = V // VT
NEG = -3.4e38


def _level_body(kk, nt, vt,
                s_ref, wt_ref, e_ref, wout_ref,
                h_ref, topv_ref, topi_ref,
                m_ref, ss_ref, cv_ref, ci_ref):
    j = pl.program_id(0)
    r = s_ref.shape[0]

    @pl.when(j == 0)
    def _init():
        z = jnp.dot(s_ref[...], wt_ref[...],
                    preferred_element_type=jnp.float32) + e_ref[...]
        h_ref[...] = jax.nn.gelu(z)
        m_ref[...] = jnp.full((r, 1), NEG, jnp.float32)
        ss_ref[...] = jnp.zeros((r, 1), jnp.float32)
        cv_ref[...] = jnp.full((r, 8), NEG, jnp.float32)
        ci_ref[...] = jnp.zeros((r, 8), jnp.int32)

    logits = jnp.dot(h_ref[...], wout_ref[...],
                     preferred_element_type=jnp.float32)
    iota = lax.broadcasted_iota(jnp.int32, (r, vt), 1)

    tm = jnp.max(logits, axis=1, keepdims=True)
    m_old = m_ref[...]
    m_new = jnp.maximum(m_old, tm)
    ss_ref[...] = (ss_ref[...] * jnp.exp(m_old - m_new)
                   + jnp.sum(jnp.exp(logits - m_new), axis=1, keepdims=True))
    m_ref[...] = m_new

    lv, li = [], []
    l = logits
    v = tm
    for t in range(kk):
        if t > 0:
            v = jnp.max(l, axis=1, keepdims=True)
        idx = jnp.min(jnp.where(l == v, iota, vt), axis=1, keepdims=True)
        lv.append(v)
        li.append(idx + j * vt)
        if t + 1 < kk:
            l = jnp.where(iota == idx, NEG, l)

    cand_v = jnp.concatenate([cv_ref[:, :kk]] + lv, axis=1)
    cand_i = jnp.concatenate([ci_ref[:, :kk]] + li, axis=1)
    cw = 2 * kk
    iota2 = lax.broadcasted_iota(jnp.int32, (r, cw), 1)
    nv_cols, ni_cols = [], []
    for t in range(kk):
        v2 = jnp.max(cand_v, axis=1, keepdims=True)
        pos = jnp.min(jnp.where(cand_v == v2, iota2, cw), axis=1, keepdims=True)
        i2 = jnp.sum(jnp.where(iota2 == pos, cand_i, 0), axis=1, keepdims=True)
        nv_cols.append(v2)
        ni_cols.append(i2)
        if t + 1 < kk:
            cand_v = jnp.where(iota2 == pos, NEG, cand_v)
    cv_ref[:, :kk] = jnp.concatenate(nv_cols, axis=1)
    ci_ref[:, :kk] = jnp.concatenate(ni_cols, axis=1)

    @pl.when(j == nt - 1)
    def _fin():
        lse = m_ref[...] + jnp.log(ss_ref[...])
        topv_ref[...] = cv_ref[:, :kk] - lse
        topi_ref[...] = ci_ref[:, :kk]


def _spec_level(s, wt, e, wout, kk, interpret=False):
    r = s.shape[0]
    return pl.pallas_call(
        functools.partial(_level_body, kk, NT, VT),
        grid=(NT,),
        in_specs=[
            pl.BlockSpec((r, D), lambda j: (0, 0)),
            pl.BlockSpec((D, D), lambda j: (0, 0)),
            pl.BlockSpec((r, D), lambda j: (0, 0)),
            pl.BlockSpec((D, VT), lambda j: (0, j)),
        ],
        out_specs=[
            pl.BlockSpec((r, D), lambda j: (0, 0)),
            pl.BlockSpec((r, kk), lambda j: (0, 0)),
            pl.BlockSpec((r, kk), lambda j: (0, 0)),
        ],
        out_shape=[
            jax.ShapeDtypeStruct((r, D), jnp.float32),
            jax.ShapeDtypeStruct((r, kk), jnp.float32),
            jax.ShapeDtypeStruct((r, kk), jnp.int32),
        ],
        scratch_shapes=[
            pltpu.VMEM((r, 1), jnp.float32),
            pltpu.VMEM((r, 1), jnp.float32),
            pltpu.VMEM((r, 8), jnp.float32),
            pltpu.VMEM((r, 8), jnp.int32),
        ],
        compiler_params=pltpu.CompilerParams(
            dimension_semantics=("arbitrary",)),
        interpret=interpret,
    )(s, wt, e, wout)


def _model_body(nt, vt, he_ref, wm_ref, wl_ref,
                h_ref, nv_ref, bv_ref, bi_ref):
    j = pl.program_id(0)
    r = he_ref.shape[0]

    @pl.when(j == 0)
    def _init():
        h_ref[...] = jax.nn.gelu(
            jnp.dot(he_ref[...], wm_ref[...],
                    preferred_element_type=jnp.float32))
        bv_ref[...] = jnp.full((r, 1), NEG, jnp.float32)
        bi_ref[...] = jnp.zeros((r, 1), jnp.int32)

    logits = lax.dot_general(h_ref[...], wl_ref[...],
                             (((1,), (1,)), ((), ())),
                             preferred_element_type=jnp.float32)
    iota = lax.broadcasted_iota(jnp.int32, (r, vt), 1)
    v = jnp.max(logits, axis=1, keepdims=True)
    idx = jnp.min(jnp.where(logits == v, iota, vt), axis=1,
                  keepdims=True) + j * vt
    upd = v > bv_ref[...]
    bv_ref[...] = jnp.where(upd, v, bv_ref[...])
    bi_ref[...] = jnp.where(upd, idx, bi_ref[...])

    @pl.when(j == nt - 1)
    def _fin():
        nv_ref[...] = bi_ref[...]


def _model_pass(he, wm, wl, interpret=False):
    r = he.shape[0]
    return pl.pallas_call(
        functools.partial(_model_body, NT, VT),
        grid=(NT,),
        in_specs=[
            pl.BlockSpec((r, D), lambda j: (0, 0)),
            pl.BlockSpec((D, D), lambda j: (0, 0)),
            pl.BlockSpec((VT, D), lambda j: (j, 0)),
        ],
        out_specs=[
            pl.BlockSpec((r, D), lambda j: (0, 0)),
            pl.BlockSpec((r, 1), lambda j: (0, 0)),
        ],
        out_shape=[
            jax.ShapeDtypeStruct((r, D), jnp.float32),
            jax.ShapeDtypeStruct((r, 1), jnp.int32),
        ],
        scratch_shapes=[
            pltpu.VMEM((r, 1), jnp.float32),
            pltpu.VMEM((r, 1), jnp.int32),
        ],
        compiler_params=pltpu.CompilerParams(
            dimension_semantics=("arbitrary",)),
        interpret=interpret,
    )(he, wm, wl)


def _verify_body(ids_ref, nv_ref, cl_ref,
                 nvsel_ref, bg_ref, nc_ref, rid_ref, ctx_ref):
    ids = ids_ref[...]
    nv = nv_ref[...]
    parts = []
    for g in range(TOPK):
        grp = ids[:, 4 * g:4 * g + 4]
        parts.append(jnp.concatenate([grp[:, 1:4], grp[:, 0:1]], axis=1))
    rolled = jnp.concatenate(parts, axis=1)
    e = (rolled == nv).astype(jnp.int32)
    ncs = []
    for g in range(TOPK):
        eg = e[:, 4 * g:4 * g + 4]
        c0 = eg[:, 0:1]
        c1 = c0 * eg[:, 1:2]
        c2 = c1 * eg[:, 2:3]
        c3 = c2 * eg[:, 3:4]
        ncs.append(jnp.clip(c0 + c1 + c2 + c3, 0, NADDS - 1))
    ncm = jnp.concatenate(ncs, axis=1)
    mx = jnp.max(ncm, axis=1, keepdims=True)
    iota5 = lax.broadcasted_iota(jnp.int32, (B, TOPK), 1)
    bg = jnp.min(jnp.where(ncm == mx, iota5, TOPK), axis=1, keepdims=True)
    nvsel = jnp.zeros((B, NADDS), jnp.int32)
    for g in range(TOPK):
        nvsel = nvsel + jnp.where(bg == g, nv[:, 4 * g:4 * g + 4], 0)
    rows = lax.broadcasted_iota(jnp.int32, (B, 1), 0)
    rid_ref[...] = rows * (TOPK * NADDS) + bg * NADDS + mx
    nvsel_ref[...] = nvsel
    bg_ref[...] = bg
    nc_ref[...] = mx
    off = lax.broadcasted_iota(jnp.int32, (B, TOPK * NADDS), 1) % NADDS
    ctx_ref[...] = cl_ref[...] - ((NADDS - 1) - off)


def _verify(ids20, nv20, cl, interpret=False):
    return pl.pallas_call(
        _verify_body,
        in_specs=[
            pl.BlockSpec((B, TOPK * NADDS), lambda: (0, 0)),
            pl.BlockSpec((B, TOPK * NADDS), lambda: (0, 0)),
            pl.BlockSpec((B, 1), lambda: (0, 0)),
        ],
        out_specs=[
            pl.BlockSpec((B, NADDS), lambda: (0, 0)),
            pl.BlockSpec((B, 1), lambda: (0, 0)),
            pl.BlockSpec((B, 1), lambda: (0, 0)),
            pl.BlockSpec((B, 1), lambda: (0, 0)),
            pl.BlockSpec((B, TOPK * NADDS), lambda: (0, 0)),
        ],
        out_shape=[
            jax.ShapeDtypeStruct((B, NADDS), jnp.int32),
            jax.ShapeDtypeStruct((B, 1), jnp.int32),
            jax.ShapeDtypeStruct((B, 1), jnp.int32),
            jax.ShapeDtypeStruct((B, 1), jnp.int32),
            jax.ShapeDtypeStruct((B, TOPK * NADDS), jnp.int32),
        ],
        interpret=interpret,
    )(ids20, nv20, cl)


def _gather_rows(table, idx):
    return jnp.take(table, idx, axis=0)


def _run(input_ids, embeds, context_lengths, spec_W, spec_emb, spec_W_out,
         model_emb, model_W, lm_head_W, interpret=False):
    toks = input_ids
    states = embeds
    scores = None
    seqs = None
    nc = 1
    for i, kk in enumerate(THRESHES):
        e = _gather_rows(spec_emb, toks)
        h, topv, topi = _spec_level(states, spec_W[i], e, spec_W_out, kk,
                                    interpret=interpret)
        tv = topv.reshape(B, nc * kk)
        ti = topi.reshape(B, nc * kk)
        scores = tv if scores is None else (
            jnp.repeat(scores, kk, axis=1) + tv)
        new_col = ti[:, :, None]
        seqs = new_col if seqs is None else jnp.concatenate(
            [jnp.repeat(seqs, kk, axis=1), new_col], axis=2)
        toks = ti.reshape(B * nc * kk)
        states = jnp.repeat(h, kk, axis=0)
        nc *= kk

    _, best = lax.top_k(scores, TOPK)
    adds = jnp.take_along_axis(seqs, best[:, :, None], axis=1)
    ids = jnp.concatenate(
        [jnp.broadcast_to(input_ids[:, None, None], (B, TOPK, 1)), adds],
        axis=-1)
    ids_flat = ids.reshape(B * TOPK * NADDS)

    he = _gather_rows(model_emb, ids_flat)
    h, nva = _model_pass(he, model_W, lm_head_W, interpret=interpret)
    nv20 = nva.reshape(B, TOPK * NADDS)
    ids20 = ids_flat.reshape(B, TOPK * NADDS)

    nvsel, bg, ncv, rid, ctx = _verify(
        ids20, nv20, context_lengths.reshape(B, 1), interpret=interpret)
    emb_final = _gather_rows(h, rid.reshape(B))[:, None, :]
    return (nvsel, emb_final, bg.reshape(B), ncv.reshape(B),
            ctx.reshape(B * TOPK * NADDS))


def kernel(input_ids, embeds, context_lengths, spec_W, spec_emb, spec_W_out,
           model_emb, model_W, lm_head_W):
    return _run(input_ids, embeds, context_lengths, spec_W, spec_emb,
                spec_W_out, model_emb, model_W, lm_head_W)
